# Initial kernel scaffold; baseline (speedup 1.0000x reference)
import jax
import jax.numpy as jnp
from jax.experimental import pallas as pl


def kernel(accumulated_times, time_encoding):
    raise NotImplementedError("write your pallas kernel here")


# speedup vs baseline: 1.8465x; 1.8465x over previous
import functools

import jax
import jax.numpy as jnp
from jax import lax
from jax.experimental import pallas as pl
from jax.experimental.pallas import tpu as pltpu
from jax.experimental.pallas import tpu_sc as plsc

_INFO = plsc.get_sparse_core_info()
_NC, _NS = _INFO.num_cores, _INFO.num_subcores
_NW = _NC * _NS

_B = 16384
_D = 1024
_BPW = _B // _NW
_C = 64
_NCHUNK = _BPW // _C


def _gather_kernel(idx_hbm, table_hbm, out_hbm, idx_v, rows_v, sem):
    wid = lax.axis_index("s") * _NC + lax.axis_index("c")
    base = wid * _BPW
    pltpu.sync_copy(idx_hbm.at[wid], idx_v)
    for c in range(_NCHUNK):
        pltpu.async_copy(table_hbm.at[idx_v.at[c]], rows_v, sem).wait()
        pltpu.sync_copy(rows_v, out_hbm.at[pl.ds(base + c * _C, _C)])


@functools.partial(jax.jit, static_argnames=())
def _run(idx_flat, table):
    mesh = plsc.VectorSubcoreMesh(core_axis_name="c", subcore_axis_name="s")
    k = pl.kernel(
        _gather_kernel,
        out_type=jax.ShapeDtypeStruct((_B, _D), jnp.float32),
        mesh=mesh,
        scratch_types=[
            pltpu.VMEM((_NCHUNK, _C), jnp.int32),
            pltpu.VMEM((_C, _D), jnp.float32),
            pltpu.SemaphoreType.DMA,
        ],
    )
    return k(idx_flat, table)


def kernel(accumulated_times, time_encoding):
    table = time_encoding.reshape(time_encoding.shape[1], time_encoding.shape[2])
    idx = accumulated_times.reshape(_NW, _NCHUNK, _C)
    out = _run(idx, table)
    return out.reshape(accumulated_times.shape + (table.shape[1],))


# speedup vs baseline: 1.9450x; 1.0533x over previous
import functools

import jax
import jax.numpy as jnp
from jax import lax
from jax.experimental import pallas as pl
from jax.experimental.pallas import tpu as pltpu
from jax.experimental.pallas import tpu_sc as plsc

_INFO = plsc.get_sparse_core_info()
_NC, _NS = _INFO.num_cores, _INFO.num_subcores
_NW = _NC * _NS

_B = 16384
_D = 1024
_BPW = _B // _NW
_C = 32
_NCHUNK = _BPW // _C
_NBUF = 3


def _gather_kernel(idx_hbm, table_hbm, out_hbm, idx_v, bufs, gsems, wsems):
    wid = lax.axis_index("s") * _NC + lax.axis_index("c")
    base = wid * _BPW
    pltpu.sync_copy(idx_hbm.at[wid], idx_v)

    def gather(c):
        return pltpu.async_copy(
            table_hbm.at[idx_v.at[c]], bufs[c % _NBUF], gsems[c % _NBUF])

    def write(c):
        return pltpu.async_copy(
            bufs[c % _NBUF], out_hbm.at[pl.ds(base + c * _C, _C)],
            wsems[c % _NBUF])

    g = [None] * _NCHUNK
    w = [None] * _NCHUNK
    g[0] = gather(0)
    g[1] = gather(1)
    for c in range(_NCHUNK):
        g[c].wait()
        w[c] = write(c)
        nc = c + 2
        if nc < _NCHUNK:
            if nc - _NBUF >= 0:
                w[nc - _NBUF].wait()
            g[nc] = gather(nc)
    for c in range(_NCHUNK - _NBUF, _NCHUNK):
        w[c].wait()


@functools.partial(jax.jit, static_argnames=())
def _run(idx_flat, table):
    mesh = plsc.VectorSubcoreMesh(core_axis_name="c", subcore_axis_name="s")
    k = pl.kernel(
        _gather_kernel,
        out_type=jax.ShapeDtypeStruct((_B, _D), jnp.float32),
        mesh=mesh,
        scratch_types=[
            pltpu.VMEM((_NCHUNK, _C), jnp.int32),
            [pltpu.VMEM((_C, _D), jnp.float32) for _ in range(_NBUF)],
            [pltpu.SemaphoreType.DMA for _ in range(_NBUF)],
            [pltpu.SemaphoreType.DMA for _ in range(_NBUF)],
        ],
    )
    return k(idx_flat, table)


def kernel(accumulated_times, time_encoding):
    table = time_encoding.reshape(time_encoding.shape[1], time_encoding.shape[2])
    idx = accumulated_times.reshape(_NW, _NCHUNK, _C)
    out = _run(idx, table)
    return out.reshape(accumulated_times.shape + (table.shape[1],))


# speedup vs baseline: 2.1095x; 1.0846x over previous
import functools

import jax
import jax.numpy as jnp
from jax import lax
from jax.experimental import pallas as pl
from jax.experimental.pallas import tpu as pltpu
from jax.experimental.pallas import tpu_sc as plsc

_INFO = plsc.get_sparse_core_info()
_NC, _NS = _INFO.num_cores, _INFO.num_subcores
_NW = _NC * _NS

_BATCH = 4
_SEQ = 4096
_D = 1024
_BPW = _BATCH * _SEQ // _NW
_WPB = _SEQ // _BPW
_C = 32
_NCHUNK = _BPW // _C
_NBUF = 3


def _gather_kernel(idx_hbm, table_hbm, out_hbm, idx_v, bufs, gsems, wsems):
    wid = lax.axis_index("s") * _NC + lax.axis_index("c")
    b = wid // _WPB
    off = (wid % _WPB) * _BPW
    pltpu.sync_copy(idx_hbm.at[b, pl.ds(off, _BPW)], idx_v)
    table2d = table_hbm.at[0]

    def gather(c):
        return pltpu.async_copy(
            table2d.at[idx_v.at[pl.ds(c * _C, _C)]],
            bufs[c % _NBUF], gsems[c % _NBUF])

    def write(c):
        return pltpu.async_copy(
            bufs[c % _NBUF], out_hbm.at[b, pl.ds(off + c * _C, _C)],
            wsems[c % _NBUF])

    g = [None] * _NCHUNK
    w = [None] * _NCHUNK
    g[0] = gather(0)
    g[1] = gather(1)
    for c in range(_NCHUNK):
        g[c].wait()
        w[c] = write(c)
        nc = c + 2
        if nc < _NCHUNK:
            if nc - _NBUF >= 0:
                w[nc - _NBUF].wait()
            g[nc] = gather(nc)
    for c in range(_NCHUNK - _NBUF, _NCHUNK):
        w[c].wait()


@jax.jit
def _run(idx, table):
    mesh = plsc.VectorSubcoreMesh(core_axis_name="c", subcore_axis_name="s")
    k = pl.kernel(
        _gather_kernel,
        out_type=jax.ShapeDtypeStruct((_BATCH, _SEQ, _D), jnp.float32),
        mesh=mesh,
        scratch_types=[
            pltpu.VMEM((_BPW,), jnp.int32),
            [pltpu.VMEM((_C, _D), jnp.float32) for _ in range(_NBUF)],
            [pltpu.SemaphoreType.DMA for _ in range(_NBUF)],
            [pltpu.SemaphoreType.DMA for _ in range(_NBUF)],
        ],
    )
    return k(idx, table)


def kernel(accumulated_times, time_encoding):
    return _run(accumulated_times, time_encoding)
